# baseline (speedup 1.0000x reference)
import functools

import jax
import jax.numpy as jnp
from jax import lax
from jax.experimental import pallas as pl
from jax.experimental.pallas import tpu as pltpu
from jax.experimental.pallas import tpu_sc as plsc

N = 100000
E = 3200000
C = 16
K = 10

NPAD = 100352
EPAD = 3211264
CHUNK = 128
BCH = 2
NBLK = 784 // BCH
NBUF = 4
ROWS_PER_TILE = NPAD // 16

_mesh = plsc.VectorSubcoreMesh(core_axis_name="c", subcore_axis_name="s")


NQUAD = NBLK // NBUF


@functools.partial(
    pl.kernel,
    mesh=_mesh,
    compiler_params=pltpu.CompilerParams(use_tc_tiling_on_sc=False),
    out_type=jax.ShapeDtypeStruct((2, NPAD, C), jnp.float32),
    scratch_types=[
        pltpu.VMEM((NBUF, BCH, CHUNK), jnp.int32),
        pltpu.VMEM((NBUF, BCH, CHUNK), jnp.int32),
        pltpu.VMEM((NBUF, BCH, CHUNK, C), jnp.float32),
        pltpu.VMEM_SHARED((NPAD, C), jnp.float32),
        pltpu.SemaphoreType.DMA,
        pltpu.SemaphoreType.DMA,
        pltpu.SemaphoreType.DMA,
        pltpu.SemaphoreType.DMA,
    ],
)
def _sc_propagate(g_hbm, src_hbm, dst_hbm, zeros_hbm, out_hbm,
                  srcv, dstv, rowsv, acc, sem0, sem1, sem2, sem3):
    cid = lax.axis_index("c")
    sid = lax.axis_index("s")
    wid = sid * 2 + cid
    r0 = sid * ROWS_PER_TILE
    sems = (sem0, sem1, sem2, sem3)
    pltpu.sync_copy(zeros_hbm.at[pl.ds(r0, ROWS_PER_TILE)],
                    acc.at[pl.ds(r0, ROWS_PER_TILE)])
    plsc.subcore_barrier()

    idxbase = wid * (NBLK * BCH)

    def fire(blk, buf):
        rowblk = idxbase + blk * BCH
        pltpu.sync_copy(src_hbm.at[pl.ds(rowblk, BCH)], srcv.at[buf])
        pltpu.sync_copy(dst_hbm.at[pl.ds(rowblk, BCH)], dstv.at[buf])
        for b in range(BCH):
            pltpu.async_copy(g_hbm.at[srcv.at[buf, b]], rowsv.at[buf, b],
                             sems[buf])

    def drain_scatter(buf):
        for b in range(BCH):
            pltpu.make_async_copy(g_hbm.at[srcv.at[buf, b]],
                                  rowsv.at[buf, b], sems[buf]).wait()
        for b in range(BCH):
            pltpu.sync_copy(rowsv.at[buf, b], acc.at[dstv.at[buf, b]],
                            add=True)

    fire(0, 0)
    fire(1, 1)
    fire(2, 2)

    def body(q, carry):
        base = q * 4
        fire(base + 3, 3)
        drain_scatter(0)
        fire(base + 4, 0)
        drain_scatter(1)
        fire(base + 5, 1)
        drain_scatter(2)
        fire(base + 6, 2)
        drain_scatter(3)
        return carry

    lax.fori_loop(0, NQUAD - 2, body, 0)
    base = (NQUAD - 2) * 4
    fire(base + 3, 3)
    drain_scatter(0)
    fire(base + 4, 0)
    drain_scatter(1)
    fire(base + 5, 1)
    drain_scatter(2)
    fire(base + 6, 2)
    drain_scatter(3)
    fire(base + 7, 3)
    drain_scatter(0)
    drain_scatter(1)
    drain_scatter(2)
    drain_scatter(3)

    plsc.subcore_barrier()
    pltpu.sync_copy(acc.at[pl.ds(r0, ROWS_PER_TILE)],
                    out_hbm.at[cid, pl.ds(r0, ROWS_PER_TILE)])


@functools.partial(
    pl.kernel,
    mesh=_mesh,
    compiler_params=pltpu.CompilerParams(use_tc_tiling_on_sc=False),
    out_type=jax.ShapeDtypeStruct((2, NPAD, C), jnp.float32),
    scratch_types=[
        pltpu.VMEM((BCH, CHUNK), jnp.int32),
        pltpu.VMEM((BCH, CHUNK, C), jnp.float32),
        pltpu.VMEM_SHARED((NPAD, C), jnp.float32),
    ],
)
def _sc_degree(ones_hbm, dst_hbm, zeros_hbm, out_hbm, dstv, onesv, acc):
    cid = lax.axis_index("c")
    sid = lax.axis_index("s")
    wid = sid * 2 + cid
    r0 = sid * ROWS_PER_TILE
    pltpu.sync_copy(zeros_hbm.at[pl.ds(r0, ROWS_PER_TILE)],
                    acc.at[pl.ds(r0, ROWS_PER_TILE)])
    pltpu.sync_copy(ones_hbm, onesv)
    plsc.subcore_barrier()

    idxbase = wid * (NBLK * BCH)

    def body(blk, carry):
        rowblk = idxbase + blk * BCH
        pltpu.sync_copy(dst_hbm.at[pl.ds(rowblk, BCH)], dstv)
        for b in range(BCH):
            pltpu.sync_copy(onesv.at[b], acc.at[dstv.at[b]], add=True)
        return carry

    lax.fori_loop(0, NBLK, body, 0)
    plsc.subcore_barrier()
    pltpu.sync_copy(acc.at[pl.ds(r0, ROWS_PER_TILE)],
                    out_hbm.at[cid, pl.ds(r0, ROWS_PER_TILE)])


_BLK = 2048
_NROWBLK = NPAD // _BLK


def _mlp_body(feats_ref, W1_ref, b1_ref, W2_ref, b2_ref, out_ref):
    x = feats_ref[...]
    h = jnp.maximum(
        jnp.dot(x, W1_ref[...], preferred_element_type=jnp.float32) + b1_ref[...],
        0.0)
    out_ref[...] = jnp.dot(h, W2_ref[...], preferred_element_type=jnp.float32) + b2_ref[...]


def _mlp(feats, W1, b1, W2, b2):
    BLK = 2000
    return pl.pallas_call(
        _mlp_body,
        grid=(N // BLK,),
        in_specs=[
            pl.BlockSpec((BLK, 128), lambda i: (i, 0)),
            pl.BlockSpec((128, 128), lambda i: (0, 0)),
            pl.BlockSpec((128,), lambda i: (0,)),
            pl.BlockSpec((128, C), lambda i: (0, 0)),
            pl.BlockSpec((C,), lambda i: (0,)),
        ],
        out_specs=pl.BlockSpec((BLK, C), lambda i: (i, 0)),
        out_shape=jax.ShapeDtypeStruct((N, C), jnp.float32),
    )(feats, W1, b1, W2, b2)


def _acc_spec():
    return [
        pl.BlockSpec((1, _BLK, C), lambda i: (0, i, 0)),
        pl.BlockSpec((1, _BLK, C), lambda i: (1, i, 0)),
    ]


def _norm_g_body(d0_ref, d1_ref, x_ref, nm_ref, g_ref):
    d = d0_ref[...][0] + d1_ref[...][0]
    nm = jnp.where(d > 0.0, lax.rsqrt(d), 0.0)
    nm_ref[...] = nm
    g_ref[...] = x_ref[...] * nm


def _norm_g(deg, x_pad):
    return pl.pallas_call(
        _norm_g_body,
        grid=(_NROWBLK,),
        in_specs=_acc_spec() + [pl.BlockSpec((_BLK, C), lambda i: (i, 0))],
        out_specs=[pl.BlockSpec((_BLK, C), lambda i: (i, 0))] * 2,
        out_shape=[jax.ShapeDtypeStruct((NPAD, C), jnp.float32)] * 2,
    )(deg, deg, x_pad)


def _scale_body(a0_ref, a1_ref, nm_ref, h_ref, g_ref):
    nm = nm_ref[...]
    h = (a0_ref[...][0] + a1_ref[...][0]) * nm
    h_ref[...] = h
    g_ref[...] = h * nm


def _scale(acc, nm):
    return pl.pallas_call(
        _scale_body,
        grid=(_NROWBLK,),
        in_specs=_acc_spec() + [pl.BlockSpec((_BLK, C), lambda i: (i, 0))],
        out_specs=[pl.BlockSpec((_BLK, C), lambda i: (i, 0))] * 2,
        out_shape=[jax.ShapeDtypeStruct((NPAD, C), jnp.float32)] * 2,
    )(acc, acc, nm)


def _combine_body(*refs):
    s_ref = refs[K + 1]
    out_ref = refs[K + 2]
    srow = s_ref[...][0:1, :]
    acc = jnp.zeros_like(out_ref[...])
    for k in range(K + 1):
        hk = refs[k][...]
        score = jax.nn.sigmoid(jnp.sum(hk * srow, axis=1, keepdims=True))
        acc = acc + score * hk
    out_ref[...] = acc


def _combine(hs, s_pad):
    return pl.pallas_call(
        _combine_body,
        grid=(_NROWBLK,),
        in_specs=[pl.BlockSpec((_BLK, C), lambda i: (i, 0))] * (K + 1)
        + [pl.BlockSpec((8, C), lambda i: (0, 0))],
        out_specs=pl.BlockSpec((_BLK, C), lambda i: (i, 0)),
        out_shape=jax.ShapeDtypeStruct((NPAD, C), jnp.float32),
    )(*hs, s_pad)


def kernel(feats, edge_index, W1, b1, W2, b2, s):
    pad = jnp.full((EPAD - E,), N, jnp.int32)
    src2 = jnp.concatenate([edge_index[0].astype(jnp.int32), pad]).reshape(EPAD // CHUNK, CHUNK)
    dst2 = jnp.concatenate([edge_index[1].astype(jnp.int32), pad]).reshape(EPAD // CHUNK, CHUNK)
    zeros = jnp.zeros((NPAD, C), jnp.float32)
    ones_blk = jnp.ones((BCH, CHUNK, C), jnp.float32)
    s_pad = jnp.zeros((8, C), jnp.float32).at[0, :].set(s[:, 0])

    x = _mlp(feats, W1, b1, W2, b2)
    x_pad = jnp.pad(x, ((0, NPAD - N), (0, 0)))

    deg = _sc_degree(ones_blk, dst2, zeros)
    nm, g = _norm_g(deg, x_pad)

    hs = [x_pad]
    for _ in range(K):
        acc = _sc_propagate(g, src2, dst2, zeros)
        h, g = _scale(acc, nm)
        hs.append(h)

    out = _combine(hs, s_pad)
    return out[:N]


# speedup vs baseline: 1.4019x; 1.4019x over previous
import functools

import jax
import jax.numpy as jnp
from jax import lax
from jax.experimental import pallas as pl
from jax.experimental.pallas import tpu as pltpu
from jax.experimental.pallas import tpu_sc as plsc

N = 100000
E = 3200000
C = 16
K = 10

NPAD = 100352
EPAD = 3211264
CHUNK = 128
BCH = 4
NBLK = 784 // BCH
ROWS_PER_TILE = NPAD // 16

_mesh = plsc.VectorSubcoreMesh(core_axis_name="c", subcore_axis_name="s")


NPAIR = NBLK // 2


@functools.partial(
    pl.kernel,
    mesh=_mesh,
    compiler_params=pltpu.CompilerParams(use_tc_tiling_on_sc=False),
    out_type=jax.ShapeDtypeStruct((2, NPAD, C), jnp.float32),
    scratch_types=[
        pltpu.VMEM((2, BCH, 2, CHUNK), jnp.int32),
        pltpu.VMEM((2, BCH, CHUNK, C), jnp.float32),
        pltpu.VMEM_SHARED((NPAD, C), jnp.float32),
        pltpu.SemaphoreType.DMA,
        pltpu.SemaphoreType.DMA,
    ],
)
def _sc_propagate(g_hbm, idx_hbm, zeros_hbm, out_hbm,
                  idxv, rowsv, acc, sem0, sem1):
    cid = lax.axis_index("c")
    sid = lax.axis_index("s")
    wid = sid * 2 + cid
    r0 = sid * ROWS_PER_TILE
    sems = (sem0, sem1)
    pltpu.sync_copy(zeros_hbm.at[pl.ds(r0, ROWS_PER_TILE)],
                    acc.at[pl.ds(r0, ROWS_PER_TILE)])
    plsc.subcore_barrier()

    idxbase = wid * (NBLK * BCH)

    def fire(blk, buf):
        rowblk = idxbase + blk * BCH
        pltpu.sync_copy(idx_hbm.at[pl.ds(rowblk, BCH)], idxv.at[buf])
        for b in range(BCH):
            pltpu.async_copy(g_hbm.at[idxv.at[buf, b, 0]], rowsv.at[buf, b],
                             sems[buf])

    def drain_scatter(buf):
        for b in range(BCH):
            pltpu.make_async_copy(g_hbm.at[idxv.at[buf, b, 0]],
                                  rowsv.at[buf, b], sems[buf]).wait()
        for b in range(BCH):
            pltpu.sync_copy(rowsv.at[buf, b], acc.at[idxv.at[buf, b, 1]],
                            add=True)

    fire(0, 0)

    def body(p, carry):
        blk = p * 2
        fire(blk + 1, 1)
        drain_scatter(0)
        fire(blk + 2, 0)
        drain_scatter(1)
        return carry

    lax.fori_loop(0, NPAIR - 1, body, 0)
    fire(NBLK - 1, 1)
    drain_scatter(0)
    drain_scatter(1)

    plsc.subcore_barrier()
    pltpu.sync_copy(acc.at[pl.ds(r0, ROWS_PER_TILE)],
                    out_hbm.at[cid, pl.ds(r0, ROWS_PER_TILE)])


@functools.partial(
    pl.kernel,
    mesh=_mesh,
    compiler_params=pltpu.CompilerParams(use_tc_tiling_on_sc=False),
    out_type=jax.ShapeDtypeStruct((2, NPAD, C), jnp.float32),
    scratch_types=[
        pltpu.VMEM((BCH, CHUNK), jnp.int32),
        pltpu.VMEM((BCH, CHUNK, C), jnp.float32),
        pltpu.VMEM_SHARED((NPAD, C), jnp.float32),
    ],
)
def _sc_degree(ones_hbm, dst_hbm, zeros_hbm, out_hbm, dstv, onesv, acc):
    cid = lax.axis_index("c")
    sid = lax.axis_index("s")
    wid = sid * 2 + cid
    r0 = sid * ROWS_PER_TILE
    pltpu.sync_copy(zeros_hbm.at[pl.ds(r0, ROWS_PER_TILE)],
                    acc.at[pl.ds(r0, ROWS_PER_TILE)])
    pltpu.sync_copy(ones_hbm, onesv)
    plsc.subcore_barrier()

    idxbase = wid * (NBLK * BCH)

    def body(blk, carry):
        rowblk = idxbase + blk * BCH
        pltpu.sync_copy(dst_hbm.at[pl.ds(rowblk, BCH)], dstv)
        for b in range(BCH):
            pltpu.sync_copy(onesv.at[b], acc.at[dstv.at[b]], add=True)
        return carry

    lax.fori_loop(0, NBLK, body, 0)
    plsc.subcore_barrier()
    pltpu.sync_copy(acc.at[pl.ds(r0, ROWS_PER_TILE)],
                    out_hbm.at[cid, pl.ds(r0, ROWS_PER_TILE)])


_BLK = 2048
_NROWBLK = NPAD // _BLK


def _mlp_body(feats_ref, W1_ref, b1_ref, W2_ref, b2_ref, out_ref):
    x = feats_ref[...]
    h = jnp.maximum(
        jnp.dot(x, W1_ref[...], preferred_element_type=jnp.float32) + b1_ref[...],
        0.0)
    out_ref[...] = jnp.dot(h, W2_ref[...], preferred_element_type=jnp.float32) + b2_ref[...]


def _mlp(feats, W1, b1, W2, b2):
    BLK = 2000
    return pl.pallas_call(
        _mlp_body,
        grid=(N // BLK,),
        in_specs=[
            pl.BlockSpec((BLK, 128), lambda i: (i, 0)),
            pl.BlockSpec((128, 128), lambda i: (0, 0)),
            pl.BlockSpec((128,), lambda i: (0,)),
            pl.BlockSpec((128, C), lambda i: (0, 0)),
            pl.BlockSpec((C,), lambda i: (0,)),
        ],
        out_specs=pl.BlockSpec((BLK, C), lambda i: (i, 0)),
        out_shape=jax.ShapeDtypeStruct((N, C), jnp.float32),
    )(feats, W1, b1, W2, b2)


def _acc_spec():
    return [
        pl.BlockSpec((1, _BLK, C), lambda i: (0, i, 0)),
        pl.BlockSpec((1, _BLK, C), lambda i: (1, i, 0)),
    ]


def _norm_g_body(d0_ref, d1_ref, x_ref, nm_ref, g_ref):
    d = d0_ref[...][0] + d1_ref[...][0]
    nm = jnp.where(d > 0.0, lax.rsqrt(d), 0.0)
    nm_ref[...] = nm
    g_ref[...] = x_ref[...] * nm


def _norm_g(deg, x_pad):
    return pl.pallas_call(
        _norm_g_body,
        grid=(_NROWBLK,),
        in_specs=_acc_spec() + [pl.BlockSpec((_BLK, C), lambda i: (i, 0))],
        out_specs=[pl.BlockSpec((_BLK, C), lambda i: (i, 0))] * 2,
        out_shape=[jax.ShapeDtypeStruct((NPAD, C), jnp.float32)] * 2,
    )(deg, deg, x_pad)


def _scale_body(a0_ref, a1_ref, nm_ref, h_ref, g_ref):
    nm = nm_ref[...]
    h = (a0_ref[...][0] + a1_ref[...][0]) * nm
    h_ref[...] = h
    g_ref[...] = h * nm


def _scale(acc, nm):
    return pl.pallas_call(
        _scale_body,
        grid=(_NROWBLK,),
        in_specs=_acc_spec() + [pl.BlockSpec((_BLK, C), lambda i: (i, 0))],
        out_specs=[pl.BlockSpec((_BLK, C), lambda i: (i, 0))] * 2,
        out_shape=[jax.ShapeDtypeStruct((NPAD, C), jnp.float32)] * 2,
    )(acc, acc, nm)


def _combine_body(*refs):
    s_ref = refs[K + 1]
    out_ref = refs[K + 2]
    srow = s_ref[...][0:1, :]
    acc = jnp.zeros_like(out_ref[...])
    for k in range(K + 1):
        hk = refs[k][...]
        score = jax.nn.sigmoid(jnp.sum(hk * srow, axis=1, keepdims=True))
        acc = acc + score * hk
    out_ref[...] = acc


def _combine(hs, s_pad):
    return pl.pallas_call(
        _combine_body,
        grid=(_NROWBLK,),
        in_specs=[pl.BlockSpec((_BLK, C), lambda i: (i, 0))] * (K + 1)
        + [pl.BlockSpec((8, C), lambda i: (0, 0))],
        out_specs=pl.BlockSpec((_BLK, C), lambda i: (i, 0)),
        out_shape=jax.ShapeDtypeStruct((NPAD, C), jnp.float32),
    )(*hs, s_pad)


def kernel(feats, edge_index, W1, b1, W2, b2, s):
    pad = jnp.full((EPAD - E,), N, jnp.int32)
    src2 = jnp.concatenate([edge_index[0].astype(jnp.int32), pad]).reshape(EPAD // CHUNK, CHUNK)
    dst2 = jnp.concatenate([edge_index[1].astype(jnp.int32), pad]).reshape(EPAD // CHUNK, CHUNK)
    idx2 = jnp.stack([src2, dst2], axis=1)
    zeros = jnp.zeros((NPAD, C), jnp.float32)
    ones_blk = jnp.ones((BCH, CHUNK, C), jnp.float32)
    s_pad = jnp.zeros((8, C), jnp.float32).at[0, :].set(s[:, 0])

    x = _mlp(feats, W1, b1, W2, b2)
    x_pad = jnp.pad(x, ((0, NPAD - N), (0, 0)))

    deg = _sc_degree(ones_blk, dst2, zeros)
    nm, g = _norm_g(deg, x_pad)

    hs = [x_pad]
    for _ in range(K):
        acc = _sc_propagate(g, idx2, zeros)
        h, g = _scale(acc, nm)
        hs.append(h)

    out = _combine(hs, s_pad)
    return out[:N]
